# Initial kernel scaffold; baseline (speedup 1.0000x reference)
import jax
import jax.numpy as jnp
from jax.experimental import pallas as pl


def kernel(xl, x0, Wg, bg, W1, b1, W2, b2):
    raise NotImplementedError("write your pallas kernel here")


# speedup vs baseline: 3.7704x; 3.7704x over previous
import functools

import jax
import jax.numpy as jnp
from jax import lax
from jax.experimental import pallas as pl
from jax.experimental.pallas import tpu as pltpu
from jax.experimental.pallas import tpu_sc as plsc

TB = 256
GATE_ROWS = 1024


def _gating_body(x_ref, wg_ref, bg_ref, out_ref):
    lt = lax.dot_general(
        wg_ref[...], x_ref[...],
        dimension_numbers=(((0,), (1,)), ((), ())),
        preferred_element_type=jnp.float32,
    ) + bg_ref[...]
    e_dim = lt.shape[0]
    iota_s = lax.broadcasted_iota(jnp.int32, lt.shape, 0)
    maxv = jnp.max(lt, axis=0, keepdims=True)
    idx = jnp.min(jnp.where(lt == maxv, iota_s, e_dim), axis=0, keepdims=True)
    out_ref[...] = idx[None].astype(jnp.int32)


def _gating(x0, wg, bg):
    n, d = x0.shape
    e = wg.shape[1]
    nrows = n // GATE_ROWS
    out = pl.pallas_call(
        _gating_body,
        grid=(nrows,),
        in_specs=[
            pl.BlockSpec((GATE_ROWS, d), lambda g: (g, 0)),
            pl.BlockSpec((d, e), lambda g: (0, 0)),
            pl.BlockSpec((e, 1), lambda g: (0, 0)),
        ],
        out_specs=pl.BlockSpec((1, 1, GATE_ROWS), lambda g: (g, 0, 0)),
        out_shape=jax.ShapeDtypeStruct((nrows, 1, GATE_ROWS), jnp.int32),
    )(x0, wg, bg.reshape(e, 1))
    return out.reshape(nrows, GATE_ROWS)


def _routing_body(ex_ref, pos_ref, be_ref, *, n_experts, n_blocks):
    ex = ex_ref[...]
    r_dim, w_dim = ex.shape
    t_lane = (lax.broadcasted_iota(jnp.int32, (w_dim, w_dim), 0)
              < lax.broadcasted_iota(jnp.int32, (w_dim, w_dim), 1)).astype(jnp.float32)
    t_row = (lax.broadcasted_iota(jnp.int32, (r_dim, r_dim), 1)
             < lax.broadcasted_iota(jnp.int32, (r_dim, r_dim), 0)).astype(jnp.float32)
    pos = jnp.zeros(ex.shape, jnp.float32)
    poff = jnp.float32(0.0)
    pends = []
    for e in range(n_experts):
        eq = (ex == e).astype(jnp.float32)
        lane_cum = lax.dot_general(eq, t_lane, (((1,), (0,)), ((), ())),
                                   preferred_element_type=jnp.float32)
        row_sums = jnp.sum(eq, axis=1, keepdims=True)
        row_cum = lax.dot_general(t_row, row_sums, (((1,), (0,)), ((), ())),
                                  preferred_element_type=jnp.float32)
        rank = lane_cum + row_cum
        cnt = jnp.sum(row_sums)
        pcnt = jnp.ceil(cnt / TB) * TB
        pos = pos + eq * (poff + rank)
        poff = poff + pcnt
        pends.append(poff)
    pos_ref[...] = pos.astype(jnp.int32)
    g_iota = lax.broadcasted_iota(jnp.int32, be_ref.shape, 1) * TB
    be = jnp.zeros(be_ref.shape, jnp.int32)
    for e in range(n_experts):
        be = be + (pends[e].astype(jnp.int32) <= g_iota).astype(jnp.int32)
    be_ref[...] = jnp.minimum(be, n_experts - 1)
    del n_blocks


def _routing(expert2d, n_experts, n_blocks):
    r_dim, w_dim = expert2d.shape
    bw = max(128, n_blocks)
    pos, be = pl.pallas_call(
        functools.partial(_routing_body, n_experts=n_experts, n_blocks=n_blocks),
        in_specs=[pl.BlockSpec((r_dim, w_dim), lambda: (0, 0))],
        out_specs=[
            pl.BlockSpec((r_dim, w_dim), lambda: (0, 0)),
            pl.BlockSpec((1, bw), lambda: (0, 0)),
        ],
        out_shape=[
            jax.ShapeDtypeStruct((r_dim, w_dim), jnp.int32),
            jax.ShapeDtypeStruct((1, bw), jnp.int32),
        ],
    )(expert2d)
    return pos, be[0, :n_blocks]


def _sc_worker_id():
    return lax.axis_index("s") * 2 + lax.axis_index("c")


def _make_dispatch(n, d, np_rows, nw, nch, chunk):
    mesh = plsc.VectorSubcoreMesh(core_axis_name="c", subcore_axis_name="s")

    @functools.partial(
        pl.kernel,
        out_type=jax.ShapeDtypeStruct((np_rows, d), jnp.float32),
        mesh=mesh,
        scratch_types=[
            pltpu.VMEM((nch, chunk), jnp.int32),
            pltpu.VMEM((chunk, d), jnp.float32),
            pltpu.SemaphoreType.DMA,
        ],
    )
    def dispatch(xl_hbm, pos3_hbm, xs_hbm, idx_v, rows_v, sem):
        w = _sc_worker_id()
        pltpu.sync_copy(pos3_hbm.at[w], idx_v)
        for j in range(nch):
            base = w * (nch * chunk) + j * chunk
            pltpu.sync_copy(xl_hbm.at[pl.ds(base, chunk)], rows_v)
            pltpu.async_copy(rows_v, xs_hbm.at[idx_v.at[j]], sem).wait()

    return dispatch


def _make_combine(n, d, np_rows, nw, nch, chunk):
    mesh = plsc.VectorSubcoreMesh(core_axis_name="c", subcore_axis_name="s")

    @functools.partial(
        pl.kernel,
        out_type=jax.ShapeDtypeStruct((n, d), jnp.float32),
        mesh=mesh,
        scratch_types=[
            pltpu.VMEM((nch, chunk), jnp.int32),
            pltpu.VMEM((chunk, d), jnp.float32),
            pltpu.SemaphoreType.DMA,
        ],
    )
    def combine(ys_hbm, pos3_hbm, out_hbm, idx_v, rows_v, sem):
        w = _sc_worker_id()
        pltpu.sync_copy(pos3_hbm.at[w], idx_v)
        for j in range(nch):
            pltpu.async_copy(ys_hbm.at[idx_v.at[j]], rows_v, sem).wait()
            base = w * (nch * chunk) + j * chunk
            pltpu.sync_copy(rows_v, out_hbm.at[pl.ds(base, chunk)])

    return combine


def _mlp_body(be_ref, x_ref, w1_ref, b1_ref, w2_ref, b2_ref, out_ref):
    del be_ref
    h = jnp.dot(x_ref[...], w1_ref[0], preferred_element_type=jnp.float32)
    h = jnp.maximum(h + b1_ref[0], 0.0)
    y = jnp.dot(h, w2_ref[0], preferred_element_type=jnp.float32)
    out_ref[...] = y + b2_ref[0]


def _grouped_mlp(xs, w1, b1, w2, b2, be):
    np_rows, d = xs.shape
    e, _, dff = w1.shape
    g = np_rows // TB
    grid_spec = pltpu.PrefetchScalarGridSpec(
        num_scalar_prefetch=1,
        grid=(g,),
        in_specs=[
            pl.BlockSpec((TB, d), lambda i, be_s: (i, 0)),
            pl.BlockSpec((1, d, dff), lambda i, be_s: (be_s[i], 0, 0)),
            pl.BlockSpec((1, 1, dff), lambda i, be_s: (be_s[i], 0, 0)),
            pl.BlockSpec((1, dff, d), lambda i, be_s: (be_s[i], 0, 0)),
            pl.BlockSpec((1, 1, d), lambda i, be_s: (be_s[i], 0, 0)),
        ],
        out_specs=pl.BlockSpec((TB, d), lambda i, be_s: (i, 0)),
    )
    return pl.pallas_call(
        _mlp_body,
        grid_spec=grid_spec,
        out_shape=jax.ShapeDtypeStruct((np_rows, d), jnp.float32),
    )(be, xs, w1, b1.reshape(e, 1, dff), w2, b2.reshape(e, 1, d))


def kernel(xl, x0, Wg, bg, W1, b1, W2, b2):
    n, d = xl.shape
    e = Wg.shape[1]
    np_rows = n + e * TB
    n_blocks = np_rows // TB
    nw = 32
    chunk = 64
    nch = n // (nw * chunk)

    expert2d = _gating(x0, Wg, bg)
    pos2d, be = _routing(expert2d, e, n_blocks)
    pos3 = pos2d.reshape(nw, nch, chunk)

    xs = _make_dispatch(n, d, np_rows, nw, nch, chunk)(xl, pos3)
    ys = _grouped_mlp(xs, W1, b1, W2, b2, be)
    out = _make_combine(n, d, np_rows, nw, nch, chunk)(ys, pos3)
    return out


# speedup vs baseline: 4.0027x; 1.0616x over previous
import functools

import jax
import jax.numpy as jnp
from jax import lax
from jax.experimental import pallas as pl
from jax.experimental.pallas import tpu as pltpu
from jax.experimental.pallas import tpu_sc as plsc

TB = 256
GATE_ROWS = 1024


def _gating_body(x_ref, wg_ref, bg_ref, out_ref):
    lt = lax.dot_general(
        wg_ref[...], x_ref[...],
        dimension_numbers=(((0,), (1,)), ((), ())),
        preferred_element_type=jnp.float32,
    ) + bg_ref[...]
    e_dim = lt.shape[0]
    iota_s = lax.broadcasted_iota(jnp.int32, lt.shape, 0)
    maxv = jnp.max(lt, axis=0, keepdims=True)
    idx = jnp.min(jnp.where(lt == maxv, iota_s, e_dim), axis=0, keepdims=True)
    out_ref[...] = idx[None].astype(jnp.int32)


def _gating(x0, wg, bg):
    n, d = x0.shape
    e = wg.shape[1]
    nrows = n // GATE_ROWS
    out = pl.pallas_call(
        _gating_body,
        grid=(nrows,),
        in_specs=[
            pl.BlockSpec((GATE_ROWS, d), lambda g: (g, 0)),
            pl.BlockSpec((d, e), lambda g: (0, 0)),
            pl.BlockSpec((e, 1), lambda g: (0, 0)),
        ],
        out_specs=pl.BlockSpec((1, 1, GATE_ROWS), lambda g: (g, 0, 0)),
        out_shape=jax.ShapeDtypeStruct((nrows, 1, GATE_ROWS), jnp.int32),
    )(x0, wg, bg.reshape(e, 1))
    return out.reshape(nrows, GATE_ROWS)


def _routing_body(ex_ref, pos_ref, be_ref, *, n_experts, n_blocks):
    ex = ex_ref[...]
    r_dim, w_dim = ex.shape
    t_lane = (lax.broadcasted_iota(jnp.int32, (w_dim, w_dim), 0)
              < lax.broadcasted_iota(jnp.int32, (w_dim, w_dim), 1)).astype(jnp.float32)
    t_row = (lax.broadcasted_iota(jnp.int32, (r_dim, r_dim), 1)
             < lax.broadcasted_iota(jnp.int32, (r_dim, r_dim), 0)).astype(jnp.float32)
    pos = jnp.zeros(ex.shape, jnp.float32)
    poff = jnp.float32(0.0)
    pends = []
    counts = []
    for e in range(n_experts):
        eq = (ex == e).astype(jnp.float32)
        lane_cum = lax.dot_general(eq, t_lane, (((1,), (0,)), ((), ())),
                                   preferred_element_type=jnp.float32)
        row_sums = jnp.sum(eq, axis=1, keepdims=True)
        row_cum = lax.dot_general(t_row, row_sums, (((1,), (0,)), ((), ())),
                                  preferred_element_type=jnp.float32)
        rank = lane_cum + row_cum
        cnt = jnp.sum(row_sums)
        pcnt = jnp.ceil(cnt / TB) * TB
        pos = pos + eq * (poff + rank)
        poff = poff + pcnt
        pends.append(poff)
        counts.append(cnt)
    pos_ref[...] = pos.astype(jnp.int32)
    g_iota = lax.broadcasted_iota(jnp.int32, (1, be_ref.shape[1]), 1) * TB
    be = jnp.zeros((1, be_ref.shape[1]), jnp.int32)
    emax = jnp.int32(0)
    for e in range(n_experts):
        pend_i = pends[e].astype(jnp.int32)
        be = be + (pend_i <= g_iota).astype(jnp.int32)
        nonzero = (counts[e] > 0).astype(jnp.int32)
        emax = jnp.maximum(emax, e * nonzero)
    total = pends[-1].astype(jnp.int32)
    valid = (g_iota < total).astype(jnp.int32)
    be_ref[...] = jnp.concatenate(
        [jnp.minimum(be, emax), valid], axis=0)
    del n_blocks


def _routing(expert2d, n_experts, n_blocks):
    r_dim, w_dim = expert2d.shape
    bw = max(128, n_blocks)
    pos, be = pl.pallas_call(
        functools.partial(_routing_body, n_experts=n_experts, n_blocks=n_blocks),
        in_specs=[pl.BlockSpec((r_dim, w_dim), lambda: (0, 0))],
        out_specs=[
            pl.BlockSpec((r_dim, w_dim), lambda: (0, 0)),
            pl.BlockSpec((2, bw), lambda: (0, 0)),
        ],
        out_shape=[
            jax.ShapeDtypeStruct((r_dim, w_dim), jnp.int32),
            jax.ShapeDtypeStruct((2, bw), jnp.int32),
        ],
    )(expert2d)
    return pos, be[0, :n_blocks], be[1, :n_blocks]


def _sc_worker_id():
    return lax.axis_index("s") * 2 + lax.axis_index("c")


def _sc_scratch(nch, chunk, d):
    return [
        pltpu.VMEM((nch, chunk), jnp.int32),
        pltpu.VMEM((2, chunk, d), jnp.float32),
        pltpu.SemaphoreType.DMA,
        pltpu.SemaphoreType.DMA,
        pltpu.SemaphoreType.DMA,
        pltpu.SemaphoreType.DMA,
    ]


def _make_dispatch(n, d, np_rows, nw, nch, chunk):
    mesh = plsc.VectorSubcoreMesh(core_axis_name="c", subcore_axis_name="s")

    @functools.partial(
        pl.kernel,
        out_type=jax.ShapeDtypeStruct((np_rows, d), jnp.float32),
        mesh=mesh,
        scratch_types=_sc_scratch(nch, chunk, d),
    )
    def dispatch(xl_hbm, pos3_hbm, xs_hbm, idx_v, rows_v, sr0, sr1, sw0, sw1):
        w = _sc_worker_id()
        sem_r, sem_w = [sr0, sr1], [sw0, sw1]
        pltpu.sync_copy(pos3_hbm.at[w], idx_v)

        def rd(j):
            base = w * (nch * chunk) + j * chunk
            return pltpu.async_copy(
                xl_hbm.at[pl.ds(base, chunk)], rows_v.at[j % 2], sem_r[j % 2])

        reads = {0: rd(0)}
        writes = {}
        for j in range(nch):
            if j + 1 < nch:
                if j - 1 >= 0:
                    writes[j - 1].wait()
                reads[j + 1] = rd(j + 1)
            reads[j].wait()
            writes[j] = pltpu.async_copy(
                rows_v.at[j % 2], xs_hbm.at[idx_v.at[j]], sem_w[j % 2])
        if nch >= 2:
            writes[nch - 2].wait()
        writes[nch - 1].wait()

    return dispatch


def _make_combine(n, d, np_rows, nw, nch, chunk):
    mesh = plsc.VectorSubcoreMesh(core_axis_name="c", subcore_axis_name="s")

    @functools.partial(
        pl.kernel,
        out_type=jax.ShapeDtypeStruct((n, d), jnp.float32),
        mesh=mesh,
        scratch_types=_sc_scratch(nch, chunk, d),
    )
    def combine(ys_hbm, pos3_hbm, out_hbm, idx_v, rows_v, sr0, sr1, sw0, sw1):
        w = _sc_worker_id()
        sem_r, sem_w = [sr0, sr1], [sw0, sw1]
        pltpu.sync_copy(pos3_hbm.at[w], idx_v)

        def rd(j):
            return pltpu.async_copy(
                ys_hbm.at[idx_v.at[j]], rows_v.at[j % 2], sem_r[j % 2])

        reads = {0: rd(0)}
        writes = {}
        for j in range(nch):
            if j + 1 < nch:
                if j - 1 >= 0:
                    writes[j - 1].wait()
                reads[j + 1] = rd(j + 1)
            reads[j].wait()
            base = w * (nch * chunk) + j * chunk
            writes[j] = pltpu.async_copy(
                rows_v.at[j % 2], out_hbm.at[pl.ds(base, chunk)], sem_w[j % 2])
        if nch >= 2:
            writes[nch - 2].wait()
        writes[nch - 1].wait()

    return combine


def _mlp_body(be_ref, valid_ref, x_ref, w1_ref, b1_ref, w2_ref, b2_ref, out_ref):
    del be_ref
    i = pl.program_id(0)

    @pl.when(valid_ref[i] == 1)
    def _():
        h = jnp.dot(x_ref[...], w1_ref[0], preferred_element_type=jnp.float32)
        h = jnp.maximum(h + b1_ref[0], 0.0)
        y = jnp.dot(h, w2_ref[0], preferred_element_type=jnp.float32)
        out_ref[...] = y + b2_ref[0]


def _grouped_mlp(xs, w1, b1, w2, b2, be, valid):
    np_rows, d = xs.shape
    e, _, dff = w1.shape
    g = np_rows // TB
    grid_spec = pltpu.PrefetchScalarGridSpec(
        num_scalar_prefetch=2,
        grid=(g,),
        in_specs=[
            pl.BlockSpec((TB, d), lambda i, be_s, v_s: (jnp.where(v_s[i] == 1, i, 0), 0)),
            pl.BlockSpec((1, d, dff), lambda i, be_s, v_s: (be_s[i], 0, 0)),
            pl.BlockSpec((1, 1, dff), lambda i, be_s, v_s: (be_s[i], 0, 0)),
            pl.BlockSpec((1, dff, d), lambda i, be_s, v_s: (be_s[i], 0, 0)),
            pl.BlockSpec((1, 1, d), lambda i, be_s, v_s: (be_s[i], 0, 0)),
        ],
        out_specs=pl.BlockSpec((TB, d), lambda i, be_s, v_s: (i, 0)),
    )
    return pl.pallas_call(
        _mlp_body,
        grid_spec=grid_spec,
        out_shape=jax.ShapeDtypeStruct((np_rows, d), jnp.float32),
    )(be, valid, xs, w1, b1.reshape(e, 1, dff), w2, b2.reshape(e, 1, d))


def kernel(xl, x0, Wg, bg, W1, b1, W2, b2):
    n, d = xl.shape
    e = Wg.shape[1]
    np_rows = n + e * TB
    n_blocks = np_rows // TB
    nw = 32
    chunk = 64
    nch = n // (nw * chunk)

    expert2d = _gating(x0, Wg, bg)
    pos2d, be, valid = _routing(expert2d, e, n_blocks)
    pos3 = pos2d.reshape(nw, nch, chunk)

    xs = _make_dispatch(n, d, np_rows, nw, nch, chunk)(xl, pos3)
    ys = _grouped_mlp(xs, W1, b1, W2, b2, be, valid)
    out = _make_combine(n, d, np_rows, nw, nch, chunk)(ys, pos3)
    return out


# speedup vs baseline: 4.3519x; 1.0873x over previous
import functools

import jax
import jax.numpy as jnp
from jax import lax
from jax.experimental import pallas as pl
from jax.experimental.pallas import tpu as pltpu
from jax.experimental.pallas import tpu_sc as plsc

TB = 512
GATE_ROWS = 1024


def _gating_body(x_ref, wg_ref, bg_ref, out_ref):
    lt = lax.dot_general(
        wg_ref[...], x_ref[...],
        dimension_numbers=(((0,), (1,)), ((), ())),
        preferred_element_type=jnp.float32,
    ) + bg_ref[...]
    e_dim = lt.shape[0]
    iota_s = lax.broadcasted_iota(jnp.int32, lt.shape, 0)
    maxv = jnp.max(lt, axis=0, keepdims=True)
    idx = jnp.min(jnp.where(lt == maxv, iota_s, e_dim), axis=0, keepdims=True)
    out_ref[...] = idx[None].astype(jnp.int32)


def _gating(x0, wg, bg):
    n, d = x0.shape
    e = wg.shape[1]
    nrows = n // GATE_ROWS
    out = pl.pallas_call(
        _gating_body,
        grid=(nrows,),
        in_specs=[
            pl.BlockSpec((GATE_ROWS, d), lambda g: (g, 0)),
            pl.BlockSpec((d, e), lambda g: (0, 0)),
            pl.BlockSpec((e, 1), lambda g: (0, 0)),
        ],
        out_specs=pl.BlockSpec((1, 1, GATE_ROWS), lambda g: (g, 0, 0)),
        out_shape=jax.ShapeDtypeStruct((nrows, 1, GATE_ROWS), jnp.int32),
    )(x0, wg, bg.reshape(e, 1))
    return out.reshape(nrows, GATE_ROWS)


def _routing_body(ex_ref, pos_ref, be_ref, *, n_experts, n_blocks):
    ex = ex_ref[...]
    r_dim, w_dim = ex.shape
    t_lane = (lax.broadcasted_iota(jnp.int32, (w_dim, w_dim), 0)
              < lax.broadcasted_iota(jnp.int32, (w_dim, w_dim), 1)).astype(jnp.float32)
    t_row = (lax.broadcasted_iota(jnp.int32, (r_dim, r_dim), 1)
             < lax.broadcasted_iota(jnp.int32, (r_dim, r_dim), 0)).astype(jnp.float32)
    pos = jnp.zeros(ex.shape, jnp.float32)
    poff = jnp.float32(0.0)
    pends = []
    counts = []
    for e in range(n_experts):
        eq = (ex == e).astype(jnp.float32)
        lane_cum = lax.dot_general(eq, t_lane, (((1,), (0,)), ((), ())),
                                   preferred_element_type=jnp.float32)
        row_sums = jnp.sum(eq, axis=1, keepdims=True)
        row_cum = lax.dot_general(t_row, row_sums, (((1,), (0,)), ((), ())),
                                  preferred_element_type=jnp.float32)
        rank = lane_cum + row_cum
        cnt = jnp.sum(row_sums)
        pcnt = jnp.ceil(cnt / TB) * TB
        pos = pos + eq * (poff + rank)
        poff = poff + pcnt
        pends.append(poff)
        counts.append(cnt)
    pos_ref[...] = pos.astype(jnp.int32)
    g_iota = lax.broadcasted_iota(jnp.int32, (1, be_ref.shape[1]), 1) * TB
    be = jnp.zeros((1, be_ref.shape[1]), jnp.int32)
    emax = jnp.int32(0)
    for e in range(n_experts):
        pend_i = pends[e].astype(jnp.int32)
        be = be + (pend_i <= g_iota).astype(jnp.int32)
        nonzero = (counts[e] > 0).astype(jnp.int32)
        emax = jnp.maximum(emax, e * nonzero)
    total = pends[-1].astype(jnp.int32)
    valid = (g_iota < total).astype(jnp.int32)
    be_ref[...] = jnp.concatenate(
        [jnp.minimum(be, emax), valid], axis=0)
    del n_blocks


def _routing(expert2d, n_experts, n_blocks):
    r_dim, w_dim = expert2d.shape
    bw = max(128, n_blocks)
    pos, be = pl.pallas_call(
        functools.partial(_routing_body, n_experts=n_experts, n_blocks=n_blocks),
        in_specs=[pl.BlockSpec((r_dim, w_dim), lambda: (0, 0))],
        out_specs=[
            pl.BlockSpec((r_dim, w_dim), lambda: (0, 0)),
            pl.BlockSpec((2, bw), lambda: (0, 0)),
        ],
        out_shape=[
            jax.ShapeDtypeStruct((r_dim, w_dim), jnp.int32),
            jax.ShapeDtypeStruct((2, bw), jnp.int32),
        ],
    )(expert2d)
    return pos, be[0, :n_blocks], be[1, :n_blocks]


def _sc_worker_id():
    return lax.axis_index("s") * 2 + lax.axis_index("c")


def _sc_scratch(nch, chunk, d):
    return [
        pltpu.VMEM((nch, chunk), jnp.int32),
        pltpu.VMEM((2, chunk, d), jnp.float32),
        pltpu.SemaphoreType.DMA,
        pltpu.SemaphoreType.DMA,
        pltpu.SemaphoreType.DMA,
        pltpu.SemaphoreType.DMA,
    ]


def _make_dispatch(n, d, np_rows, nw, nch, chunk):
    mesh = plsc.VectorSubcoreMesh(core_axis_name="c", subcore_axis_name="s")

    @functools.partial(
        pl.kernel,
        out_type=jax.ShapeDtypeStruct((np_rows, d), jnp.float32),
        mesh=mesh,
        scratch_types=_sc_scratch(nch, chunk, d),
    )
    def dispatch(xl_hbm, pos3_hbm, xs_hbm, idx_v, rows_v, sr0, sr1, sw0, sw1):
        w = _sc_worker_id()
        sem_r, sem_w = [sr0, sr1], [sw0, sw1]
        pltpu.sync_copy(pos3_hbm.at[w], idx_v)

        def rd(j):
            base = w * (nch * chunk) + j * chunk
            return pltpu.async_copy(
                xl_hbm.at[pl.ds(base, chunk)], rows_v.at[j % 2], sem_r[j % 2])

        reads = {0: rd(0)}
        writes = {}
        for j in range(nch):
            if j + 1 < nch:
                if j - 1 >= 0:
                    writes[j - 1].wait()
                reads[j + 1] = rd(j + 1)
            reads[j].wait()
            writes[j] = pltpu.async_copy(
                rows_v.at[j % 2], xs_hbm.at[idx_v.at[j]], sem_w[j % 2])
        if nch >= 2:
            writes[nch - 2].wait()
        writes[nch - 1].wait()

    return dispatch


def _make_combine(n, d, np_rows, nw, nch, chunk):
    mesh = plsc.VectorSubcoreMesh(core_axis_name="c", subcore_axis_name="s")

    @functools.partial(
        pl.kernel,
        out_type=jax.ShapeDtypeStruct((n, d), jnp.float32),
        mesh=mesh,
        scratch_types=_sc_scratch(nch, chunk, d),
    )
    def combine(ys_hbm, pos3_hbm, out_hbm, idx_v, rows_v, sr0, sr1, sw0, sw1):
        w = _sc_worker_id()
        sem_r, sem_w = [sr0, sr1], [sw0, sw1]
        pltpu.sync_copy(pos3_hbm.at[w], idx_v)

        def rd(j):
            return pltpu.async_copy(
                ys_hbm.at[idx_v.at[j]], rows_v.at[j % 2], sem_r[j % 2])

        reads = {0: rd(0)}
        writes = {}
        for j in range(nch):
            if j + 1 < nch:
                if j - 1 >= 0:
                    writes[j - 1].wait()
                reads[j + 1] = rd(j + 1)
            reads[j].wait()
            base = w * (nch * chunk) + j * chunk
            writes[j] = pltpu.async_copy(
                rows_v.at[j % 2], out_hbm.at[pl.ds(base, chunk)], sem_w[j % 2])
        if nch >= 2:
            writes[nch - 2].wait()
        writes[nch - 1].wait()

    return combine


def _mlp_body(be_ref, valid_ref, x_ref, w1_ref, b1_ref, w2_ref, b2_ref, out_ref):
    i = pl.program_id(0)

    @pl.when(valid_ref[i] == 1)
    def _():
        e_idx = be_ref[i]
        h = jnp.dot(x_ref[...], w1_ref[0], preferred_element_type=jnp.float32)
        h = jnp.maximum(h + b1_ref[pl.ds(e_idx, 1), :], 0.0)
        y = jnp.dot(h, w2_ref[0], preferred_element_type=jnp.float32)
        out_ref[...] = y + b2_ref[pl.ds(e_idx, 1), :]


def _grouped_mlp(xs, w1, b1, w2, b2, be, valid):
    np_rows, d = xs.shape
    e, _, dff = w1.shape
    g = np_rows // TB
    grid_spec = pltpu.PrefetchScalarGridSpec(
        num_scalar_prefetch=2,
        grid=(g,),
        in_specs=[
            pl.BlockSpec((TB, d), lambda i, be_s, v_s: (jnp.where(v_s[i] == 1, i, 0), 0)),
            pl.BlockSpec((1, d, dff), lambda i, be_s, v_s: (be_s[i], 0, 0)),
            pl.BlockSpec((e, dff), lambda i, be_s, v_s: (0, 0)),
            pl.BlockSpec((1, dff, d), lambda i, be_s, v_s: (be_s[i], 0, 0)),
            pl.BlockSpec((e, d), lambda i, be_s, v_s: (0, 0)),
        ],
        out_specs=pl.BlockSpec((TB, d), lambda i, be_s, v_s: (i, 0)),
    )
    return pl.pallas_call(
        _mlp_body,
        grid_spec=grid_spec,
        out_shape=jax.ShapeDtypeStruct((np_rows, d), jnp.float32),
    )(be, valid, xs, w1, b1, w2, b2)


def kernel(xl, x0, Wg, bg, W1, b1, W2, b2):
    n, d = xl.shape
    e = Wg.shape[1]
    np_rows = n + e * TB
    n_blocks = np_rows // TB
    nw = 32
    chunk = 64
    nch = n // (nw * chunk)

    expert2d = _gating(x0, Wg, bg)
    pos2d, be, valid = _routing(expert2d, e, n_blocks)
    pos3 = pos2d.reshape(nw, nch, chunk)

    xs = _make_dispatch(n, d, np_rows, nw, nch, chunk)(xl, pos3)
    ys = _grouped_mlp(xs, W1, b1, W2, b2, be, valid)
    out = _make_combine(n, d, np_rows, nw, nch, chunk)(ys, pos3)
    return out


# speedup vs baseline: 4.4397x; 1.0202x over previous
import functools

import jax
import jax.numpy as jnp
from jax import lax
from jax.experimental import pallas as pl
from jax.experimental.pallas import tpu as pltpu
from jax.experimental.pallas import tpu_sc as plsc

TB = 512
GATE_ROWS = 1024


def _gating_body(x_ref, wg_ref, bg_ref, out_ref):
    lt = lax.dot_general(
        wg_ref[...], x_ref[...],
        dimension_numbers=(((0,), (1,)), ((), ())),
        preferred_element_type=jnp.float32,
    ) + bg_ref[...]
    e_dim = lt.shape[0]
    iota_s = lax.broadcasted_iota(jnp.int32, lt.shape, 0)
    maxv = jnp.max(lt, axis=0, keepdims=True)
    idx = jnp.min(jnp.where(lt == maxv, iota_s, e_dim), axis=0, keepdims=True)
    out_ref[...] = idx[None].astype(jnp.int32)


def _gating(x0, wg, bg):
    n, d = x0.shape
    e = wg.shape[1]
    nrows = n // GATE_ROWS
    out = pl.pallas_call(
        _gating_body,
        grid=(nrows,),
        in_specs=[
            pl.BlockSpec((GATE_ROWS, d), lambda g: (g, 0)),
            pl.BlockSpec((d, e), lambda g: (0, 0)),
            pl.BlockSpec((e, 1), lambda g: (0, 0)),
        ],
        out_specs=pl.BlockSpec((1, 1, GATE_ROWS), lambda g: (g, 0, 0)),
        out_shape=jax.ShapeDtypeStruct((nrows, 1, GATE_ROWS), jnp.int32),
    )(x0, wg, bg.reshape(e, 1))
    return out


def _routing_body(ex_ref, pos_ref, be_ref, *, n_experts, n_blocks):
    ex = ex_ref[...][:, 0, :]
    r_dim, w_dim = ex.shape
    t_lane = (lax.broadcasted_iota(jnp.int32, (w_dim, w_dim), 0)
              < lax.broadcasted_iota(jnp.int32, (w_dim, w_dim), 1)).astype(jnp.float32)
    t_row = (lax.broadcasted_iota(jnp.int32, (r_dim, r_dim), 1)
             < lax.broadcasted_iota(jnp.int32, (r_dim, r_dim), 0)).astype(jnp.float32)
    pos = jnp.zeros(ex.shape, jnp.float32)
    poff = jnp.float32(0.0)
    pends = []
    counts = []
    for e in range(n_experts):
        eq = (ex == e).astype(jnp.float32)
        lane_cum = lax.dot_general(eq, t_lane, (((1,), (0,)), ((), ())),
                                   preferred_element_type=jnp.float32)
        row_sums = jnp.sum(eq, axis=1, keepdims=True)
        row_cum = lax.dot_general(t_row, row_sums, (((1,), (0,)), ((), ())),
                                  preferred_element_type=jnp.float32)
        rank = lane_cum + row_cum
        cnt = jnp.sum(row_sums)
        pcnt = jnp.ceil(cnt / TB) * TB
        pos = pos + eq * (poff + rank)
        poff = poff + pcnt
        pends.append(poff)
        counts.append(cnt)
    pos_ref[...] = pos.astype(jnp.int32)
    g_iota = lax.broadcasted_iota(jnp.int32, (1, be_ref.shape[1]), 1) * TB
    be = jnp.zeros((1, be_ref.shape[1]), jnp.int32)
    emax = jnp.int32(0)
    for e in range(n_experts):
        pend_i = pends[e].astype(jnp.int32)
        be = be + (pend_i <= g_iota).astype(jnp.int32)
        nonzero = (counts[e] > 0).astype(jnp.int32)
        emax = jnp.maximum(emax, e * nonzero)
    total = pends[-1].astype(jnp.int32)
    valid = (g_iota < total).astype(jnp.int32)
    be_ref[...] = jnp.concatenate(
        [jnp.minimum(be, emax), valid], axis=0)
    del n_blocks


def _routing(expert3d, n_experts, n_blocks):
    r_dim, _, w_dim = expert3d.shape
    bw = max(128, n_blocks)
    pos, be2 = pl.pallas_call(
        functools.partial(_routing_body, n_experts=n_experts, n_blocks=n_blocks),
        in_specs=[pl.BlockSpec((r_dim, 1, w_dim), lambda: (0, 0, 0))],
        out_specs=[
            pl.BlockSpec((r_dim, w_dim), lambda: (0, 0)),
            pl.BlockSpec((2, bw), lambda: (0, 0)),
        ],
        out_shape=[
            jax.ShapeDtypeStruct((r_dim, w_dim), jnp.int32),
            jax.ShapeDtypeStruct((2, bw), jnp.int32),
        ],
    )(expert3d)
    return pos, be2


def _sc_worker_id():
    return lax.axis_index("s") * 2 + lax.axis_index("c")


NBUF = 4


def _sc_scratch(nch, chunk, d):
    return [
        pltpu.VMEM((nch, chunk), jnp.int32),
        pltpu.VMEM((NBUF, chunk, d), jnp.float32),
    ] + [pltpu.SemaphoreType.DMA] * (2 * NBUF)


def _ring(nch, rd, wr):
    reads, writes = {}, {}
    for j in range(min(NBUF - 1, nch)):
        reads[j] = rd(j)
    for j in range(nch):
        nxt = j + NBUF - 1
        if nxt < nch:
            prev = nxt - NBUF
            if prev >= 0:
                writes.pop(prev).wait()
            reads[nxt] = rd(nxt)
        reads[j].wait()
        writes[j] = wr(j)
    for j in sorted(writes):
        writes[j].wait()


def _make_dispatch(n, d, np_rows, nw, nch, chunk):
    mesh = plsc.VectorSubcoreMesh(core_axis_name="c", subcore_axis_name="s")

    @functools.partial(
        pl.kernel,
        out_type=jax.ShapeDtypeStruct((np_rows, d), jnp.float32),
        mesh=mesh,
        scratch_types=_sc_scratch(nch, chunk, d),
    )
    def dispatch(xl_hbm, pos3_hbm, xs_hbm, idx_v, rows_v, *sems):
        w = _sc_worker_id()
        sem_r, sem_w = sems[:NBUF], sems[NBUF:]
        pltpu.sync_copy(pos3_hbm.at[w], idx_v)

        def rd(j):
            base = w * (nch * chunk) + j * chunk
            return pltpu.async_copy(
                xl_hbm.at[pl.ds(base, chunk)], rows_v.at[j % NBUF], sem_r[j % NBUF])

        def wr(j):
            return pltpu.async_copy(
                rows_v.at[j % NBUF], xs_hbm.at[idx_v.at[j]], sem_w[j % NBUF])

        _ring(nch, rd, wr)

    return dispatch


def _make_combine(n, d, np_rows, nw, nch, chunk):
    mesh = plsc.VectorSubcoreMesh(core_axis_name="c", subcore_axis_name="s")

    @functools.partial(
        pl.kernel,
        out_type=jax.ShapeDtypeStruct((n, d), jnp.float32),
        mesh=mesh,
        scratch_types=_sc_scratch(nch, chunk, d),
    )
    def combine(ys_hbm, pos3_hbm, out_hbm, idx_v, rows_v, *sems):
        w = _sc_worker_id()
        sem_r, sem_w = sems[:NBUF], sems[NBUF:]
        pltpu.sync_copy(pos3_hbm.at[w], idx_v)

        def rd(j):
            return pltpu.async_copy(
                ys_hbm.at[idx_v.at[j]], rows_v.at[j % NBUF], sem_r[j % NBUF])

        def wr(j):
            base = w * (nch * chunk) + j * chunk
            return pltpu.async_copy(
                rows_v.at[j % NBUF], out_hbm.at[pl.ds(base, chunk)], sem_w[j % NBUF])

        _ring(nch, rd, wr)

    return combine


def _mlp_body(meta_ref, x_ref, w1_ref, b1_ref, w2_ref, b2_ref, out_ref):
    i = pl.program_id(0)

    @pl.when(meta_ref[1, i] == 1)
    def _():
        e_idx = meta_ref[0, i]
        h = jnp.dot(x_ref[...], w1_ref[0], preferred_element_type=jnp.float32)
        h = jnp.maximum(h + b1_ref[pl.ds(e_idx, 1), :], 0.0)
        y = jnp.dot(h, w2_ref[0], preferred_element_type=jnp.float32)
        out_ref[...] = y + b2_ref[pl.ds(e_idx, 1), :]


def _grouped_mlp(xs, w1, b1, w2, b2, meta):
    np_rows, d = xs.shape
    e, _, dff = w1.shape
    g = np_rows // TB
    grid_spec = pltpu.PrefetchScalarGridSpec(
        num_scalar_prefetch=1,
        grid=(g,),
        in_specs=[
            pl.BlockSpec((TB, d), lambda i, m_s: (jnp.where(m_s[1, i] == 1, i, 0), 0)),
            pl.BlockSpec((1, d, dff), lambda i, m_s: (m_s[0, i], 0, 0)),
            pl.BlockSpec((e, dff), lambda i, m_s: (0, 0)),
            pl.BlockSpec((1, dff, d), lambda i, m_s: (m_s[0, i], 0, 0)),
            pl.BlockSpec((e, d), lambda i, m_s: (0, 0)),
        ],
        out_specs=pl.BlockSpec((TB, d), lambda i, m_s: (i, 0)),
    )
    return pl.pallas_call(
        _mlp_body,
        grid_spec=grid_spec,
        out_shape=jax.ShapeDtypeStruct((np_rows, d), jnp.float32),
    )(meta, xs, w1, b1, w2, b2)


def kernel(xl, x0, Wg, bg, W1, b1, W2, b2):
    n, d = xl.shape
    e = Wg.shape[1]
    np_rows = n + e * TB
    n_blocks = np_rows // TB
    nw = 32
    chunk = 32
    nch = n // (nw * chunk)

    expert3d = _gating(x0, Wg, bg)
    pos2d, meta = _routing(expert3d, e, n_blocks)
    pos3 = pos2d.reshape(nw, nch, chunk)

    xs = _make_dispatch(n, d, np_rows, nw, nch, chunk)(xl, pos3)
    ys = _grouped_mlp(xs, W1, b1, W2, b2, meta)
    out = _make_combine(n, d, np_rows, nw, nch, chunk)(ys, pos3)
    return out
